# Initial kernel scaffold; baseline (speedup 1.0000x reference)
import jax
import jax.numpy as jnp
from jax.experimental import pallas as pl


def kernel(x, edge_index, W1, b1, W2, b2):
    raise NotImplementedError("write your pallas kernel here")


# speedup vs baseline: 18.2870x; 18.2870x over previous
import functools

import jax
import jax.numpy as jnp
from jax import lax
from jax.experimental import pallas as pl
from jax.experimental.pallas import tpu as pltpu
from jax.experimental.pallas import tpu_sc as plsc

N = 10000
D_IN = 128
D_HID = 16
NCLS = 40
D2P = 48
D_CNT = 16

NC = 2
NS = 16
NW = NC * NS
CH = 128

E_RAW = 320000
E_PAD = ((E_RAW + NW * CH - 1) // (NW * CH)) * (NW * CH)
E_PER_TILE = E_PAD // NW
CHUNKS = E_PER_TILE // CH

N_PAD = 10112
ROWS_PER_TILE = N_PAD // NS


def _make_agg(d, gather):
  mesh = plsc.VectorSubcoreMesh(core_axis_name="c", subcore_axis_name="s")

  @functools.partial(
      pl.kernel,
      out_type=jax.ShapeDtypeStruct((NC, N_PAD, d), jnp.float32),
      mesh=mesh,
      scratch_types=[
          pltpu.VMEM((CH,), jnp.int32),
          pltpu.VMEM((CH,), jnp.int32),
          pltpu.VMEM((CH, d), jnp.float32),
          pltpu.VMEM_SHARED((N_PAD, d), jnp.float32),
          pltpu.SemaphoreType.DMA,
      ],
      compiler_params=pltpu.CompilerParams(use_tc_tiling_on_sc=False),
  )
  def k(table_hbm, src_hbm, dst_hbm, zeros_hbm, out_hbm,
        src_v, dst_v, rows_v, acc_sh, sem):
    c = lax.axis_index("c")
    s = lax.axis_index("s")
    wid = s * NC + c
    zbase = s * ROWS_PER_TILE
    pltpu.sync_copy(zeros_hbm.at[pl.ds(zbase, ROWS_PER_TILE)],
                    acc_sh.at[pl.ds(zbase, ROWS_PER_TILE)])
    plsc.subcore_barrier()
    if not gather:
      pltpu.sync_copy(table_hbm, rows_v)
    ebase = wid * E_PER_TILE

    def body(i, carry):
      off = ebase + i * CH
      pltpu.sync_copy(dst_hbm.at[pl.ds(off, CH)], dst_v)
      if gather:
        pltpu.sync_copy(src_hbm.at[pl.ds(off, CH)], src_v)
        pltpu.async_copy(table_hbm.at[src_v], rows_v, sem).wait()
      pltpu.sync_copy(rows_v, acc_sh.at[dst_v], add=True)
      return carry

    lax.fori_loop(0, CHUNKS, body, 0)
    plsc.subcore_barrier()
    pltpu.sync_copy(acc_sh.at[pl.ds(zbase, ROWS_PER_TILE)],
                    out_hbm.at[c, pl.ds(zbase, ROWS_PER_TILE)])

  return k


_agg_cnt = _make_agg(D_CNT, gather=False)
_agg_l1 = _make_agg(D_HID, gather=True)
_agg_l2 = _make_agg(D2P, gather=True)


def _mm1_body(x_ref, w_ref, o_ref):
  o_ref[...] = jnp.dot(x_ref[...], w_ref[...],
                       preferred_element_type=jnp.float32)


def _scale1_body(cnt_ref, xw_ref, xws_ref, dis_ref):
  deg = cnt_ref[0, 0:N, 0:1] + cnt_ref[1, 0:N, 0:1] + 1.0
  dis = lax.rsqrt(deg)
  dis_ref[...] = dis
  xws_ref[...] = xw_ref[...] * dis


def _mid_body(agg_ref, xws_ref, dis_ref, b1_ref, w2_ref, o_ref):
  aggs = agg_ref[0, 0:N, :] + agg_ref[1, 0:N, :]
  h = jnp.maximum(dis_ref[...] * (aggs + xws_ref[...]) + b1_ref[...], 0.0)
  o_ref[...] = jnp.dot(h, w2_ref[...],
                       preferred_element_type=jnp.float32) * dis_ref[...]


def _fin_body(agg_ref, hws_ref, dis_ref, b2_ref, o_ref):
  aggs = agg_ref[0, 0:N, :] + agg_ref[1, 0:N, :]
  z = dis_ref[...] * (aggs + hws_ref[...]) + b2_ref[...]
  m = jnp.max(z, axis=1, keepdims=True)
  lse = jnp.log(jnp.sum(jnp.exp(z - m), axis=1, keepdims=True)) + m
  o_ref[...] = z - lse


def kernel(x, edge_index, W1, b1, W2, b2):
  src = edge_index[0].astype(jnp.int32)
  dst = edge_index[1].astype(jnp.int32)
  pad = E_PAD - src.shape[0]
  src_p = jnp.concatenate([src, jnp.zeros((pad,), jnp.int32)])
  dst_p = jnp.concatenate([dst, jnp.full((pad,), N, jnp.int32)])

  ones_tab = jnp.ones((CH, D_CNT), jnp.float32)
  zeros_cnt = jnp.zeros((N_PAD, D_CNT), jnp.float32)
  cnt = _agg_cnt(ones_tab, src_p, dst_p, zeros_cnt)

  xw1 = pl.pallas_call(
      _mm1_body,
      out_shape=jax.ShapeDtypeStruct((N, D_HID), jnp.float32),
  )(x, W1)

  xws1, dis = pl.pallas_call(
      _scale1_body,
      out_shape=(
          jax.ShapeDtypeStruct((N, D_HID), jnp.float32),
          jax.ShapeDtypeStruct((N, 1), jnp.float32),
      ),
  )(cnt, xw1)

  zeros_l1 = jnp.zeros((N_PAD, D_HID), jnp.float32)
  agg1 = _agg_l1(xws1, src_p, dst_p, zeros_l1)

  w2p = jnp.pad(W2, ((0, 0), (0, D2P - NCLS)))
  b1r = b1.reshape(1, D_HID)
  hws = pl.pallas_call(
      _mid_body,
      out_shape=jax.ShapeDtypeStruct((N, D2P), jnp.float32),
  )(agg1, xws1, dis, b1r, w2p)

  zeros_l2 = jnp.zeros((N_PAD, D2P), jnp.float32)
  agg2 = _agg_l2(hws, src_p, dst_p, zeros_l2)

  b2p = jnp.pad(b2.reshape(1, NCLS), ((0, 0), (0, D2P - NCLS)),
                constant_values=-1e30)
  out48 = pl.pallas_call(
      _fin_body,
      out_shape=jax.ShapeDtypeStruct((N, D2P), jnp.float32),
  )(agg2, hws, dis, b2p)
  return out48[:, :NCLS]
